# Initial kernel scaffold; baseline (speedup 1.0000x reference)
import jax
import jax.numpy as jnp
from jax.experimental import pallas as pl


def kernel(X_nucl, We, position_encoding):
    raise NotImplementedError("write your pallas kernel here")


# speedup vs baseline: 4.8155x; 4.8155x over previous
import jax
import jax.numpy as jnp
from jax import lax
from jax.experimental import pallas as pl
from jax.experimental.pallas import tpu as pltpu
from jax.experimental.pallas import tpu_sc as plsc

NC = 2
NS = 16
NW = NC * NS

B, L, E, K = 16384, 200, 4, 5
IDX_PER_W = B * L // NW
ROWS_PER_CHUNK = 8
CHUNK_I = ROWS_PER_CHUNK * L
CHUNK_O = CHUNK_I * E
NCHUNKS = IDX_PER_W // CHUNK_I
VECS_PER_ROW = L * E // 16


def _sc_body(x_hbm, t_hbm, out_hbm, t_v, idx_v, out_v):
    wid = lax.axis_index("s") * NC + lax.axis_index("c")
    pltpu.sync_copy(t_hbm, t_v)

    iota = lax.iota(jnp.int32, 16)
    quad = iota >> 2
    stat = quad * 20 + (iota & 3)

    def chunk_body(c, carry):
        base = wid * IDX_PER_W + c * CHUNK_I
        pltpu.sync_copy(x_hbm.at[pl.ds(base, CHUNK_I)], idx_v)

        def row_body(r, carry):
            def vec_body(v, carry):
                lidx = r * L + v * 4 + quad
                idx16 = plsc.load_gather(idx_v, [lidx])
                addr = idx16 * 4 + (stat + v * 80)
                val = plsc.load_gather(t_v, [addr])
                out_v[pl.ds(r * (L * E) + v * 16, 16)] = val
                return carry

            return lax.fori_loop(0, VECS_PER_ROW, vec_body, carry)

        lax.fori_loop(0, ROWS_PER_CHUNK, row_body, 0)
        pltpu.sync_copy(out_v, out_hbm.at[pl.ds(base * E, CHUNK_O)])
        return carry

    lax.fori_loop(0, NCHUNKS, chunk_body, 0)


def kernel(X_nucl, We, position_encoding):
    x_flat = X_nucl.astype(jnp.int32).reshape(-1)
    t = (position_encoding[0][:, None, :] + We[None, :, :]).reshape(-1)
    t = t.astype(jnp.float32)

    call = pl.kernel(
        _sc_body,
        out_type=jax.ShapeDtypeStruct((B * L * E,), jnp.float32),
        mesh=plsc.VectorSubcoreMesh(core_axis_name="c", subcore_axis_name="s"),
        compiler_params=pltpu.CompilerParams(needs_layout_passes=False),
        scratch_types=[
            pltpu.VMEM((L * K * E,), jnp.float32),
            pltpu.VMEM((CHUNK_I,), jnp.int32),
            pltpu.VMEM((CHUNK_O,), jnp.float32),
        ],
    )
    out = call(x_flat, t)
    return out.reshape(B, L, E)


# speedup vs baseline: 5.3056x; 1.1018x over previous
import jax
import jax.numpy as jnp
from jax import lax
from jax.experimental import pallas as pl
from jax.experimental.pallas import tpu as pltpu
from jax.experimental.pallas import tpu_sc as plsc

NC = 2
NS = 16
NW = NC * NS

B, L, E, K = 16384, 200, 4, 5
IDX_PER_W = B * L // NW
ROWS_PER_CHUNK = 8
CHUNK_I = ROWS_PER_CHUNK * L
CHUNK_O = CHUNK_I * E
NCHUNKS = IDX_PER_W // CHUNK_I
VECS_PER_ROW = L * E // 16


def _sc_body(x_hbm, t_hbm, out_hbm, t_v, idx_v, out_v):
    wid = lax.axis_index("s") * NC + lax.axis_index("c")
    pltpu.sync_copy(t_hbm, t_v)

    iota = lax.iota(jnp.int32, 16)
    quad = iota >> 2
    stat = quad * 20 + (iota & 3)

    def chunk_body(c, carry):
        base = wid * IDX_PER_W + c * CHUNK_I
        pltpu.sync_copy(x_hbm.at[pl.ds(base, CHUNK_I)], idx_v)

        def row_body(r, carry):
            @plsc.parallel_loop(0, VECS_PER_ROW, unroll=10)
            def vec_body(v):
                lidx = r * L + v * 4 + quad
                idx16 = plsc.load_gather(idx_v, [lidx])
                addr = (idx16 << 2) + (stat + v * 80)
                val = plsc.load_gather(t_v, [addr])
                out_v[pl.ds(r * (L * E) + v * 16, 16)] = val

            return carry

        lax.fori_loop(0, ROWS_PER_CHUNK, row_body, 0)
        pltpu.sync_copy(out_v, out_hbm.at[pl.ds(base * E, CHUNK_O)])
        return carry

    lax.fori_loop(0, NCHUNKS, chunk_body, 0)


def kernel(X_nucl, We, position_encoding):
    x_flat = X_nucl.astype(jnp.int32).reshape(-1)
    t = (position_encoding[0][:, None, :] + We[None, :, :]).reshape(-1)
    t = t.astype(jnp.float32)

    call = pl.kernel(
        _sc_body,
        out_type=jax.ShapeDtypeStruct((B * L * E,), jnp.float32),
        mesh=plsc.VectorSubcoreMesh(core_axis_name="c", subcore_axis_name="s"),
        compiler_params=pltpu.CompilerParams(needs_layout_passes=False),
        scratch_types=[
            pltpu.VMEM((L * K * E,), jnp.float32),
            pltpu.VMEM((CHUNK_I,), jnp.int32),
            pltpu.VMEM((CHUNK_O,), jnp.float32),
        ],
    )
    out = call(x_flat, t)
    return out.reshape(B, L, E)


# speedup vs baseline: 44.2892x; 8.3477x over previous
import jax
import jax.numpy as jnp
from jax import lax
from jax.experimental import pallas as pl
from jax.experimental.pallas import tpu as pltpu
from jax.experimental.pallas import tpu_sc as plsc

NC = 2
NS = 16
NW = NC * NS

B, L, E, K = 16384, 200, 4, 5
ROWS_PER_W = B // NW
ROWS_PER_CHUNK = 8
NCHUNKS = ROWS_PER_W // ROWS_PER_CHUNK
VECS_PER_ROW = L * E // 16


def _sc_body(x_hbm, t_hbm, out_hbm, t_v, idx_v, out_v):
    wid = lax.axis_index("s") * NC + lax.axis_index("c")
    pltpu.sync_copy(t_hbm, t_v)

    iota = lax.iota(jnp.int32, 16)
    quad = iota >> 2
    stat = quad * 20 + (iota & 3)

    def chunk_body(c, carry):
        row0 = wid * ROWS_PER_W + c * ROWS_PER_CHUNK
        pltpu.sync_copy(x_hbm.at[pl.ds(row0, ROWS_PER_CHUNK)], idx_v)

        def row_body(r, carry):
            rvec = jnp.full((16,), r, jnp.int32)

            @plsc.parallel_loop(0, VECS_PER_ROW, unroll=10)
            def vec_body(v):
                colv = v * 4 + quad
                idx16 = plsc.load_gather(idx_v, [rvec, colv])
                addr = (idx16 << 2) + (stat + v * 80)
                val = plsc.load_gather(t_v, [addr])
                out_v[r, pl.ds(v * 16, 16)] = val

            return carry

        lax.fori_loop(0, ROWS_PER_CHUNK, row_body, 0)
        pltpu.sync_copy(out_v, out_hbm.at[pl.ds(row0, ROWS_PER_CHUNK)])
        return carry

    lax.fori_loop(0, NCHUNKS, chunk_body, 0)


def kernel(X_nucl, We, position_encoding):
    x = X_nucl.astype(jnp.int32)
    t = (position_encoding[0][:, None, :] + We[None, :, :]).reshape(-1)
    t = t.astype(jnp.float32)

    call = pl.kernel(
        _sc_body,
        out_type=jax.ShapeDtypeStruct((B, L * E), jnp.float32),
        mesh=plsc.VectorSubcoreMesh(core_axis_name="c", subcore_axis_name="s"),
        compiler_params=pltpu.CompilerParams(needs_layout_passes=False),
        scratch_types=[
            pltpu.VMEM((L * K * E,), jnp.float32),
            pltpu.VMEM((ROWS_PER_CHUNK, L), jnp.int32),
            pltpu.VMEM((ROWS_PER_CHUNK, L * E), jnp.float32),
        ],
    )
    out = call(x, t)
    return out.reshape(B, L, E)


# speedup vs baseline: 53.5724x; 1.2096x over previous
import jax
import jax.numpy as jnp
from jax import lax
from jax.experimental import pallas as pl
from jax.experimental.pallas import tpu as pltpu
from jax.experimental.pallas import tpu_sc as plsc

NC = 2
NS = 16
NW = NC * NS

B, L, E, K = 16384, 200, 4, 5
ROWS_PER_W = B // NW
CH = 32
NCHUNKS = ROWS_PER_W // CH
VECS_PER_ROW = L * E // 16


def _sc_body(x_hbm, t_hbm, out_hbm, t_v, idx0, idx1, out0, out1,
             sin0, sin1, sout0, sout1):
    wid = lax.axis_index("s") * NC + lax.axis_index("c")
    pltpu.sync_copy(t_hbm, t_v)

    iota = lax.iota(jnp.int32, 16)
    quad = iota >> 2
    epat = iota & 3
    stat = quad * 20 + epat

    idxb, outb = (idx0, idx1), (out0, out1)
    sins, souts = (sin0, sin1), (sout0, sout1)

    def row0(c):
        return wid * ROWS_PER_W + c * CH

    def compute(idx_v, out_v):
        def row_body(r, carry):
            rvec = jnp.full((16,), r, jnp.int32)

            @plsc.parallel_loop(0, VECS_PER_ROW, unroll=10)
            def vec_body(v):
                colv = v * 4 + quad
                idx16 = plsc.load_gather(idx_v, [rvec, colv])
                addr = (idx16 << 2) + (stat + v * 80)
                val = plsc.load_gather(t_v, [addr])
                out_v[r, pl.ds(v * 16, 16)] = val

            return carry

        lax.fori_loop(0, CH, row_body, 0)

    pltpu.make_async_copy(x_hbm.at[pl.ds(row0(0), CH)], idx0, sin0).start()
    for c in range(NCHUNKS):
        b = c & 1
        pltpu.make_async_copy(x_hbm.at[pl.ds(row0(c), CH)], idxb[b], sins[b]).wait()
        if c + 1 < NCHUNKS:
            pltpu.make_async_copy(
                x_hbm.at[pl.ds(row0(c + 1), CH)], idxb[1 - b], sins[1 - b]
            ).start()
        if c >= 2:
            pltpu.make_async_copy(
                outb[b], out_hbm.at[pl.ds(row0(c - 2), CH)], souts[b]
            ).wait()
        compute(idxb[b], outb[b])
        pltpu.make_async_copy(outb[b], out_hbm.at[pl.ds(row0(c), CH)], souts[b]).start()

    for c in (NCHUNKS - 2, NCHUNKS - 1):
        b = c & 1
        pltpu.make_async_copy(outb[b], out_hbm.at[pl.ds(row0(c), CH)], souts[b]).wait()


def kernel(X_nucl, We, position_encoding):
    x = X_nucl.astype(jnp.int32)
    t = (position_encoding[0][:, None, :] + We[None, :, :]).reshape(-1)
    t = t.astype(jnp.float32)

    call = pl.kernel(
        _sc_body,
        out_type=jax.ShapeDtypeStruct((B, L * E), jnp.float32),
        mesh=plsc.VectorSubcoreMesh(core_axis_name="c", subcore_axis_name="s"),
        compiler_params=pltpu.CompilerParams(needs_layout_passes=False),
        scratch_types=[
            pltpu.VMEM((L * K * E,), jnp.float32),
            pltpu.VMEM((CH, L), jnp.int32),
            pltpu.VMEM((CH, L), jnp.int32),
            pltpu.VMEM((CH, L * E), jnp.float32),
            pltpu.VMEM((CH, L * E), jnp.float32),
            pltpu.SemaphoreType.DMA,
            pltpu.SemaphoreType.DMA,
            pltpu.SemaphoreType.DMA,
            pltpu.SemaphoreType.DMA,
        ],
    )
    out = call(x, t)
    return out.reshape(B, L, E)
